# baseline (speedup 1.0000x reference)
import functools

import jax
import jax.numpy as jnp
from jax import lax
from jax.experimental import pallas as pl
from jax.experimental.pallas import tpu as pltpu
from jax.experimental.pallas import tpu_sc as plsc

N = 16384
BATCH = 256
NCHUNK = 8
CW = BATCH // NCHUNK
NCORES = 2
NSUB = 16
CPC = NCHUNK // NCORES
K = 128
ROWS_PER_TILE = N // NSUB


def _spmm_body(nblk, xc, rows, cols, vals, out, idxc2, idxr2, valv2, gbuf,
               gbuf1, gbuf2, sbuf, sbuf1, sbuf2, zbuf, acc, sem, sem1, sem2,
               sems0, sems1, sems2):
    c = lax.axis_index("c")
    s = lax.axis_index("s")
    per_tile = nblk * K

    zero16 = jnp.zeros((16,), jnp.float32)

    def zrow(i, carry):
        for w in range(CW // 16):
            zbuf[i, pl.ds(w * 16, 16)] = zero16
        return carry

    lax.fori_loop(0, K, zrow, 0)

    pltpu.sync_copy(rows.at[s], idxr2)
    pltpu.sync_copy(vals.at[s], valv2)

    for jl in range(CPC):
        j = c * CPC + jl
        for z in range(ROWS_PER_TILE // K):
            pltpu.sync_copy(zbuf, acc.at[pl.ds(s * ROWS_PER_TILE + z * K, K)])

        pltpu.sync_copy(cols.at[s], idxc2)
        col_off = j * N

        def adj(i, carry):
            idxc2[i // 8, pl.ds((i % 8) * 16, 16)] = (
                idxc2[i // 8, pl.ds((i % 8) * 16, 16)] + col_off)
            return carry

        lax.fori_loop(0, nblk * (K // 16), adj, 0)
        plsc.subcore_barrier()

        def scale(b, buf, sb):
            def rowscale(g, rc):
                v16 = valv2[b, pl.ds(g * 16, 16)]
                for l in range(16):
                    vl = v16[l]
                    i = g * 16 + l
                    xb = buf[i, pl.ds(0, CW)]
                    lo, hi = plsc.unpack(xb, format=plsc.PackFormat.INTERLEAVED)
                    sb[i, pl.ds(0, 16)] = lo * vl
                    sb[i, pl.ds(16, 16)] = hi * vl
                return rc

            lax.fori_loop(0, K // 16, rowscale, 0)

        bufs = (gbuf, gbuf1, gbuf2)
        sbufs = (sbuf, sbuf1, sbuf2)
        gsem = (sem, sem1, sem2)
        ssem = (sems0, sems1, sems2)
        pltpu.async_copy(xc.at[idxc2.at[0]], bufs[0], gsem[0])
        pltpu.async_copy(xc.at[idxc2.at[1]], bufs[1], gsem[1])

        def blk3(t, carry):
            for u in range(3):
                b = 3 * t + u
                up = (u + 2) % 3
                pltpu.make_async_copy(xc.at[idxc2.at[b]], bufs[u],
                                      gsem[u]).wait()
                scale(b, bufs[u], sbufs[u])
                pltpu.async_copy(sbufs[u], acc.at[idxr2.at[b]], ssem[u],
                                 add=True)

                @pl.when(b >= 1)
                def _():
                    pltpu.make_async_copy(sbufs[up], acc.at[idxr2.at[b - 1]],
                                          ssem[up]).wait()

                @pl.when(b + 2 < nblk)
                def _():
                    pltpu.async_copy(xc.at[idxc2.at[b + 2]], bufs[up],
                                     gsem[up])
            return carry

        lax.fori_loop(0, nblk // 3, blk3, 0)
        pltpu.make_async_copy(sbufs[(nblk - 1) % 3],
                              acc.at[idxr2.at[nblk - 1]],
                              ssem[(nblk - 1) % 3]).wait()
        plsc.subcore_barrier()
        pltpu.sync_copy(acc.at[pl.ds(s * ROWS_PER_TILE, ROWS_PER_TILE)],
                        out.at[j, pl.ds(s * ROWS_PER_TILE, ROWS_PER_TILE)])
        plsc.subcore_barrier()


def kernel(X, S_rows, S_cols, S_vals):
    nnz = S_rows.shape[0]
    per_tile = -(-nnz // NSUB)
    per_tile = -(-per_tile // (3 * K)) * (3 * K)
    nblk = per_tile // K
    pad = per_tile * NSUB - nnz
    rows_p = jnp.pad(S_rows, (0, pad)).reshape(NSUB, nblk, K)
    cols_p = jnp.pad(S_cols, (0, pad)).reshape(NSUB, nblk, K)
    vals_p = jnp.pad(S_vals, (0, pad)).reshape(NSUB, nblk, K)
    perm = jnp.arange(CW).reshape(CW // 16, 16).T.reshape(CW)
    xc = (X.reshape(NCHUNK, CW, N).transpose(0, 2, 1)[:, :, perm]
          .astype(jnp.bfloat16).reshape(NCHUNK * N, CW))

    mesh = plsc.VectorSubcoreMesh(core_axis_name="c", subcore_axis_name="s",
                                  num_cores=NCORES, num_subcores=NSUB)
    f = pl.kernel(
        functools.partial(_spmm_body, nblk),
        out_type=jax.ShapeDtypeStruct((NCHUNK, N, CW), jnp.float32),
        mesh=mesh,
        scratch_types=[
            pltpu.VMEM((nblk, K), jnp.int32),
            pltpu.VMEM((nblk, K), jnp.int32),
            pltpu.VMEM((nblk, K), jnp.float32),
            pltpu.VMEM((K, CW), jnp.bfloat16),
            pltpu.VMEM((K, CW), jnp.bfloat16),
            pltpu.VMEM((K, CW), jnp.bfloat16),
            pltpu.VMEM((K, CW), jnp.float32),
            pltpu.VMEM((K, CW), jnp.float32),
            pltpu.VMEM((K, CW), jnp.float32),
            pltpu.VMEM((K, CW), jnp.float32),
            pltpu.VMEM_SHARED((N, CW), jnp.float32),
            pltpu.SemaphoreType.DMA,
            pltpu.SemaphoreType.DMA,
            pltpu.SemaphoreType.DMA,
            pltpu.SemaphoreType.DMA,
            pltpu.SemaphoreType.DMA,
            pltpu.SemaphoreType.DMA,
        ],
        compiler_params=pltpu.CompilerParams(use_tc_tiling_on_sc=False,
                                             needs_layout_passes=False),
    )
    out_c = f(xc, rows_p, cols_p, vals_p)
    return out_c.transpose(0, 2, 1).reshape(BATCH, N)


# speedup vs baseline: 1.0210x; 1.0210x over previous
import functools

import jax
import jax.numpy as jnp
from jax import lax
from jax.experimental import pallas as pl
from jax.experimental.pallas import tpu as pltpu
from jax.experimental.pallas import tpu_sc as plsc

N = 16384
BATCH = 256
NCHUNK = 8
CW = BATCH // NCHUNK
NCORES = 2
NSUB = 16
CPC = NCHUNK // NCORES
K = 128
ROWS_PER_TILE = N // NSUB


def _spmm_body(nblk, xc, rows, cols, vals, out, idxc2, idxr2, valv2, gbuf,
               gbuf1, gbuf2, sbuf, sbuf1, sbuf2, zbuf, acc, sem, sem1, sem2,
               sems0, sems1, sems2):
    c = lax.axis_index("c")
    s = lax.axis_index("s")
    per_tile = nblk * K

    zero16 = jnp.zeros((16,), jnp.float32)

    def zrow(i, carry):
        for w in range(CW // 16):
            zbuf[i, pl.ds(w * 16, 16)] = zero16
        return carry

    lax.fori_loop(0, K, zrow, 0)

    pltpu.sync_copy(rows.at[s], idxr2)
    pltpu.sync_copy(vals.at[s], valv2)

    for jl in range(CPC):
        j = c * CPC + jl
        for z in range(ROWS_PER_TILE // K):
            pltpu.sync_copy(zbuf, acc.at[pl.ds(s * ROWS_PER_TILE + z * K, K)])

        pltpu.sync_copy(cols.at[s], idxc2)
        col_off = j * N

        def adj(i, carry):
            idxc2[i // 8, pl.ds((i % 8) * 16, 16)] = (
                idxc2[i // 8, pl.ds((i % 8) * 16, 16)] + col_off)
            return carry

        lax.fori_loop(0, nblk * (K // 16), adj, 0)
        plsc.subcore_barrier()

        def scale(b, buf, sb):
            def rowscale(g, rc):
                v16 = valv2[b, pl.ds(g * 16, 16)]
                for l in range(16):
                    vl = v16[l]
                    i = g * 16 + l
                    xb = buf[i, pl.ds(0, CW)]
                    lo, hi = plsc.unpack(xb, format=plsc.PackFormat.INTERLEAVED)
                    sb[i, pl.ds(0, 16)] = lo * vl
                    sb[i, pl.ds(16, 16)] = hi * vl
                return rc

            lax.fori_loop(0, K // 16, rowscale, 0)

        bufs = (gbuf, gbuf1, gbuf2)
        sbufs = (sbuf, sbuf1, sbuf2)
        gsem = (sem, sem1, sem2)
        ssem = (sems0, sems1, sems2)
        pltpu.async_copy(xc.at[idxc2.at[0]], bufs[0], gsem[0])
        pltpu.async_copy(xc.at[idxc2.at[1]], bufs[1], gsem[1])

        def blk3(t, carry):
            for u in range(3):
                b = 3 * t + u
                up = (u + 2) % 3
                pltpu.make_async_copy(xc.at[idxc2.at[b]], bufs[u],
                                      gsem[u]).wait()

                @pl.when(b + 2 < nblk)
                def _():
                    pltpu.async_copy(xc.at[idxc2.at[b + 2]], bufs[up],
                                     gsem[up])

                @pl.when(b >= 3)
                def _():
                    pltpu.make_async_copy(sbufs[u], acc.at[idxr2.at[b - 3]],
                                          ssem[u]).wait()

                scale(b, bufs[u], sbufs[u])
                pltpu.async_copy(sbufs[u], acc.at[idxr2.at[b]], ssem[u],
                                 add=True)
            return carry

        lax.fori_loop(0, nblk // 3, blk3, 0)
        for q in range(3):
            bq = nblk - 3 + q
            pltpu.make_async_copy(sbufs[bq % 3], acc.at[idxr2.at[bq]],
                                  ssem[bq % 3]).wait()
        plsc.subcore_barrier()
        pltpu.sync_copy(acc.at[pl.ds(s * ROWS_PER_TILE, ROWS_PER_TILE)],
                        out.at[j, pl.ds(s * ROWS_PER_TILE, ROWS_PER_TILE)])
        plsc.subcore_barrier()


def kernel(X, S_rows, S_cols, S_vals):
    nnz = S_rows.shape[0]
    per_tile = -(-nnz // NSUB)
    per_tile = -(-per_tile // (3 * K)) * (3 * K)
    nblk = per_tile // K
    pad = per_tile * NSUB - nnz
    rows_p = jnp.pad(S_rows, (0, pad)).reshape(NSUB, nblk, K)
    cols_p = jnp.pad(S_cols, (0, pad)).reshape(NSUB, nblk, K)
    vals_p = jnp.pad(S_vals, (0, pad)).reshape(NSUB, nblk, K)
    perm = jnp.arange(CW).reshape(CW // 16, 16).T.reshape(CW)
    xc = (X.reshape(NCHUNK, CW, N).transpose(0, 2, 1)[:, :, perm]
          .astype(jnp.bfloat16).reshape(NCHUNK * N, CW))

    mesh = plsc.VectorSubcoreMesh(core_axis_name="c", subcore_axis_name="s",
                                  num_cores=NCORES, num_subcores=NSUB)
    f = pl.kernel(
        functools.partial(_spmm_body, nblk),
        out_type=jax.ShapeDtypeStruct((NCHUNK, N, CW), jnp.float32),
        mesh=mesh,
        scratch_types=[
            pltpu.VMEM((nblk, K), jnp.int32),
            pltpu.VMEM((nblk, K), jnp.int32),
            pltpu.VMEM((nblk, K), jnp.float32),
            pltpu.VMEM((K, CW), jnp.bfloat16),
            pltpu.VMEM((K, CW), jnp.bfloat16),
            pltpu.VMEM((K, CW), jnp.bfloat16),
            pltpu.VMEM((K, CW), jnp.float32),
            pltpu.VMEM((K, CW), jnp.float32),
            pltpu.VMEM((K, CW), jnp.float32),
            pltpu.VMEM((K, CW), jnp.float32),
            pltpu.VMEM_SHARED((N, CW), jnp.float32),
            pltpu.SemaphoreType.DMA,
            pltpu.SemaphoreType.DMA,
            pltpu.SemaphoreType.DMA,
            pltpu.SemaphoreType.DMA,
            pltpu.SemaphoreType.DMA,
            pltpu.SemaphoreType.DMA,
        ],
        compiler_params=pltpu.CompilerParams(use_tc_tiling_on_sc=False,
                                             needs_layout_passes=False),
    )
    out_c = f(xc, rows_p, cols_p, vals_p)
    return out_c.transpose(0, 2, 1).reshape(BATCH, N)


# speedup vs baseline: 1.2337x; 1.2083x over previous
import functools

import jax
import jax.numpy as jnp
from jax import lax
from jax.experimental import pallas as pl
from jax.experimental.pallas import tpu as pltpu
from jax.experimental.pallas import tpu_sc as plsc

N = 16384
BATCH = 256
NCHUNK = 8
CW = BATCH // NCHUNK
NCORES = 2
NSUB = 16
CPC = NCHUNK // NCORES
K = 128
ROWS_PER_TILE = N // NSUB


def _spmm_body(nblk, xc, rows, cols, vals, out, idxc2, idxr2, valv2, gbuf,
               gbuf1, gbuf2, zbuf, acc, sem, sem1, sem2, sems0, sems1, sems2):
    c = lax.axis_index("c")
    s = lax.axis_index("s")
    per_tile = nblk * K

    zero16 = jnp.zeros((16,), jnp.float32)

    def zrow(i, carry):
        for w in range(CW // 16):
            zbuf[i, pl.ds(w * 16, 16)] = zero16
        return carry

    lax.fori_loop(0, K, zrow, 0)

    pltpu.sync_copy(rows.at[s], idxr2)
    pltpu.sync_copy(vals.at[s], valv2)

    for jl in range(CPC):
        j = c * CPC + jl
        for z in range(ROWS_PER_TILE // K):
            pltpu.sync_copy(zbuf, acc.at[pl.ds(s * ROWS_PER_TILE + z * K, K)])

        pltpu.sync_copy(cols.at[s], idxc2)
        col_off = j * N

        def adj(i, carry):
            idxc2[i // 8, pl.ds((i % 8) * 16, 16)] = (
                idxc2[i // 8, pl.ds((i % 8) * 16, 16)] + col_off)
            return carry

        lax.fori_loop(0, nblk * (K // 16), adj, 0)
        plsc.subcore_barrier()

        def scale(b, buf):
            def rowscale(g, rc):
                v16 = valv2[b, pl.ds(g * 16, 16)]
                for l in range(16):
                    vl = v16[l]
                    i = g * 16 + l
                    for w in range(CW // 16):
                        buf[i, pl.ds(w * 16, 16)] = (
                            buf[i, pl.ds(w * 16, 16)] * vl)
                return rc

            lax.fori_loop(0, K // 16, rowscale, 0)

        bufs = (gbuf, gbuf1, gbuf2)
        gsem = (sem, sem1, sem2)
        ssem = (sems0, sems1, sems2)
        def issue_gather(b, buf, sm):
            pltpu.async_copy(xc.at[idxc2.at[b].at[pl.ds(0, K // 2)]],
                             buf.at[pl.ds(0, K // 2)], sm)
            pltpu.async_copy(xc.at[idxc2.at[b].at[pl.ds(K // 2, K // 2)]],
                             buf.at[pl.ds(K // 2, K // 2)], sm)

        def wait_gather(b, buf, sm):
            pltpu.make_async_copy(xc.at[idxc2.at[b]], buf, sm).wait()

        issue_gather(0, bufs[0], gsem[0])
        issue_gather(1, bufs[1], gsem[1])

        def blk3(t, carry):
            for u in range(3):
                b = 3 * t + u
                up = (u + 2) % 3
                wait_gather(b, bufs[u], gsem[u])
                scale(b, bufs[u])
                pltpu.async_copy(bufs[u], acc.at[idxr2.at[b]], ssem[u],
                                 add=True)

                @pl.when(b >= 1)
                def _():
                    pltpu.make_async_copy(bufs[up], acc.at[idxr2.at[b - 1]],
                                          ssem[up]).wait()

                @pl.when(b + 2 < nblk)
                def _():
                    issue_gather(b + 2, bufs[up], gsem[up])
            return carry

        lax.fori_loop(0, nblk // 3, blk3, 0)
        pltpu.make_async_copy(bufs[(nblk - 1) % 3],
                              acc.at[idxr2.at[nblk - 1]],
                              ssem[(nblk - 1) % 3]).wait()
        plsc.subcore_barrier()
        pltpu.sync_copy(acc.at[pl.ds(s * ROWS_PER_TILE, ROWS_PER_TILE)],
                        out.at[j, pl.ds(s * ROWS_PER_TILE, ROWS_PER_TILE)])
        plsc.subcore_barrier()


def kernel(X, S_rows, S_cols, S_vals):
    nnz = S_rows.shape[0]
    per_tile = -(-nnz // NSUB)
    per_tile = -(-per_tile // (3 * K)) * (3 * K)
    nblk = per_tile // K
    pad = per_tile * NSUB - nnz
    rows_p = jnp.pad(S_rows, (0, pad)).reshape(NSUB, nblk, K)
    cols_p = jnp.pad(S_cols, (0, pad)).reshape(NSUB, nblk, K)
    vals_p = jnp.pad(S_vals, (0, pad)).reshape(NSUB, nblk, K)
    xc = X.reshape(NCHUNK, CW, N).transpose(0, 2, 1).reshape(NCHUNK * N, CW)

    mesh = plsc.VectorSubcoreMesh(core_axis_name="c", subcore_axis_name="s",
                                  num_cores=NCORES, num_subcores=NSUB)
    f = pl.kernel(
        functools.partial(_spmm_body, nblk),
        out_type=jax.ShapeDtypeStruct((NCHUNK, N, CW), jnp.float32),
        mesh=mesh,
        scratch_types=[
            pltpu.VMEM((nblk, K), jnp.int32),
            pltpu.VMEM((nblk, K), jnp.int32),
            pltpu.VMEM((nblk, K), jnp.float32),
            pltpu.VMEM((K, CW), jnp.float32),
            pltpu.VMEM((K, CW), jnp.float32),
            pltpu.VMEM((K, CW), jnp.float32),
            pltpu.VMEM((K, CW), jnp.float32),
            pltpu.VMEM_SHARED((N, CW), jnp.float32),
            pltpu.SemaphoreType.DMA,
            pltpu.SemaphoreType.DMA,
            pltpu.SemaphoreType.DMA,
            pltpu.SemaphoreType.DMA,
            pltpu.SemaphoreType.DMA,
            pltpu.SemaphoreType.DMA,
        ],
        compiler_params=pltpu.CompilerParams(use_tc_tiling_on_sc=False),
    )
    out_c = f(xc, rows_p, cols_p, vals_p)
    return out_c.transpose(0, 2, 1).reshape(BATCH, N)


# speedup vs baseline: 1.2976x; 1.0518x over previous
import functools

import jax
import jax.numpy as jnp
from jax import lax
from jax.experimental import pallas as pl
from jax.experimental.pallas import tpu as pltpu
from jax.experimental.pallas import tpu_sc as plsc

N = 16384
BATCH = 256
NCHUNK = 8
CW = BATCH // NCHUNK
NCORES = 2
NSUB = 16
CPC = NCHUNK // NCORES
K = 128
ROWS_PER_TILE = N // NSUB


def _spmm_body(nblk, xc, rows, cols, vals, out, idxc2, idxr2, valv2, gbuf,
               gbuf1, gbuf2, zbuf, acc, sem, sem1, sem2, sems0, sems1, sems2):
    c = lax.axis_index("c")
    s = lax.axis_index("s")
    per_tile = nblk * K

    zero16 = jnp.zeros((16,), jnp.float32)

    def zrow(i, carry):
        for w in range(CW // 16):
            zbuf[i, pl.ds(w * 16, 16)] = zero16
        return carry

    lax.fori_loop(0, K, zrow, 0)

    pltpu.sync_copy(rows.at[s], idxr2)
    pltpu.sync_copy(vals.at[s], valv2)
    pltpu.sync_copy(cols.at[s], idxc2)

    for jl in range(CPC):
        j = c * CPC + jl
        for z in range(ROWS_PER_TILE // K):
            pltpu.sync_copy(zbuf, acc.at[pl.ds(s * ROWS_PER_TILE + z * K, K)])

        xcj = xc.at[pl.ds(j * N, N)]
        plsc.subcore_barrier()

        def scale(b, buf):
            def rowscale(g, rc):
                v16 = valv2[b, pl.ds(g * 16, 16)]
                for l in range(16):
                    vl = v16[l]
                    i = g * 16 + l
                    for w in range(CW // 16):
                        buf[i, pl.ds(w * 16, 16)] = (
                            buf[i, pl.ds(w * 16, 16)] * vl)
                return rc

            lax.fori_loop(0, K // 16, rowscale, 0)

        bufs = (gbuf, gbuf1, gbuf2)
        gsem = (sem, sem1, sem2)
        ssem = (sems0, sems1, sems2)
        pltpu.async_copy(xcj.at[idxc2.at[0]], bufs[0], gsem[0])
        pltpu.async_copy(xcj.at[idxc2.at[1]], bufs[1], gsem[1])

        def blk3(t, carry):
            for u in range(3):
                b = 3 * t + u
                up = (u + 2) % 3
                pltpu.make_async_copy(xcj.at[idxc2.at[b]], bufs[u],
                                      gsem[u]).wait()
                scale(b, bufs[u])
                pltpu.async_copy(bufs[u], acc.at[idxr2.at[b]], ssem[u],
                                 add=True)

                @pl.when(b >= 1)
                def _():
                    pltpu.make_async_copy(bufs[up], acc.at[idxr2.at[b - 1]],
                                          ssem[up]).wait()

                @pl.when(b + 2 < nblk)
                def _():
                    pltpu.async_copy(xcj.at[idxc2.at[b + 2]], bufs[up],
                                     gsem[up])
            return carry

        lax.fori_loop(0, nblk // 3, blk3, 0)
        pltpu.make_async_copy(bufs[(nblk - 1) % 3],
                              acc.at[idxr2.at[nblk - 1]],
                              ssem[(nblk - 1) % 3]).wait()
        plsc.subcore_barrier()
        pltpu.sync_copy(acc.at[pl.ds(s * ROWS_PER_TILE, ROWS_PER_TILE)],
                        out.at[j, pl.ds(s * ROWS_PER_TILE, ROWS_PER_TILE)])
        plsc.subcore_barrier()


def kernel(X, S_rows, S_cols, S_vals):
    nnz = S_rows.shape[0]
    per_tile = -(-nnz // NSUB)
    per_tile = -(-per_tile // (3 * K)) * (3 * K)
    nblk = per_tile // K
    pad = per_tile * NSUB - nnz
    rows_p = jnp.pad(S_rows, (0, pad)).reshape(NSUB, nblk, K)
    cols_p = jnp.pad(S_cols, (0, pad)).reshape(NSUB, nblk, K)
    vals_p = jnp.pad(S_vals, (0, pad)).reshape(NSUB, nblk, K)
    xc = X.reshape(NCHUNK, CW, N).transpose(0, 2, 1).reshape(NCHUNK * N, CW)

    mesh = plsc.VectorSubcoreMesh(core_axis_name="c", subcore_axis_name="s",
                                  num_cores=NCORES, num_subcores=NSUB)
    f = pl.kernel(
        functools.partial(_spmm_body, nblk),
        out_type=jax.ShapeDtypeStruct((NCHUNK, N, CW), jnp.float32),
        mesh=mesh,
        scratch_types=[
            pltpu.VMEM((nblk, K), jnp.int32),
            pltpu.VMEM((nblk, K), jnp.int32),
            pltpu.VMEM((nblk, K), jnp.float32),
            pltpu.VMEM((K, CW), jnp.float32),
            pltpu.VMEM((K, CW), jnp.float32),
            pltpu.VMEM((K, CW), jnp.float32),
            pltpu.VMEM((K, CW), jnp.float32),
            pltpu.VMEM_SHARED((N, CW), jnp.float32),
            pltpu.SemaphoreType.DMA,
            pltpu.SemaphoreType.DMA,
            pltpu.SemaphoreType.DMA,
            pltpu.SemaphoreType.DMA,
            pltpu.SemaphoreType.DMA,
            pltpu.SemaphoreType.DMA,
        ],
        compiler_params=pltpu.CompilerParams(use_tc_tiling_on_sc=False),
    )
    out_c = f(xc, rows_p, cols_p, vals_p)
    return out_c.transpose(0, 2, 1).reshape(BATCH, N)


# speedup vs baseline: 1.3035x; 1.0046x over previous
import functools

import jax
import jax.numpy as jnp
from jax import lax
from jax.experimental import pallas as pl
from jax.experimental.pallas import tpu as pltpu
from jax.experimental.pallas import tpu_sc as plsc

N = 16384
BATCH = 256
NCHUNK = 8
CW = BATCH // NCHUNK
NCORES = 2
NSUB = 16
CPC = NCHUNK // NCORES
K = 128
ROWS_PER_TILE = N // NSUB


def _spmm_body(nblk, xc, rows, cols, vals, out, idxc2, idxr2, valv2, gbuf,
               gbuf1, gbuf2, zbuf, acc, sem, sem1, sem2, sems0, sems1, sems2):
    c = lax.axis_index("c")
    s = lax.axis_index("s")
    per_tile = nblk * K

    zero16 = jnp.zeros((16,), jnp.float32)

    def zrow(i, carry):
        for w in range(CW // 16):
            zbuf[i, pl.ds(w * 16, 16)] = zero16
        return carry

    lax.fori_loop(0, K, zrow, 0)

    pltpu.sync_copy(rows.at[s], idxr2)
    pltpu.sync_copy(vals.at[s], valv2)
    pltpu.sync_copy(cols.at[s], idxc2)

    def chunk_view(jl):
        return xc.at[pl.ds((c * CPC + jl) * N, N)]

    for z in range(ROWS_PER_TILE // K):
        pltpu.sync_copy(zbuf, acc.at[pl.ds(s * ROWS_PER_TILE + z * K, K)])
    plsc.subcore_barrier()

    for jl in range(CPC):
        j = c * CPC + jl
        xcj = chunk_view(jl)

        def scale(b, buf):
            def rowscale(g, rc):
                v16 = valv2[b, pl.ds(g * 16, 16)]
                for l in range(16):
                    vl = v16[l]
                    i = g * 16 + l
                    for w in range(CW // 16):
                        buf[i, pl.ds(w * 16, 16)] = (
                            buf[i, pl.ds(w * 16, 16)] * vl)
                return rc

            lax.fori_loop(0, K // 16, rowscale, 0)

        bufs = (gbuf, gbuf1, gbuf2)
        gsem = (sem, sem1, sem2)
        ssem = (sems0, sems1, sems2)
        if jl == 0:
            pltpu.async_copy(xcj.at[idxc2.at[0]], bufs[0], gsem[0])
            pltpu.async_copy(xcj.at[idxc2.at[1]], bufs[1], gsem[1])

        def blk3(t, carry):
            for u in range(3):
                b = 3 * t + u
                up = (u + 2) % 3
                pltpu.make_async_copy(xcj.at[idxc2.at[b]], bufs[u],
                                      gsem[u]).wait()
                scale(b, bufs[u])
                pltpu.async_copy(bufs[u], acc.at[idxr2.at[b]], ssem[u],
                                 add=True)

                @pl.when(b >= 1)
                def _():
                    pltpu.make_async_copy(bufs[up], acc.at[idxr2.at[b - 1]],
                                          ssem[up]).wait()

                @pl.when(b + 2 < nblk)
                def _():
                    pltpu.async_copy(xcj.at[idxc2.at[b + 2]], bufs[up],
                                     gsem[up])
            return carry

        lax.fori_loop(0, nblk // 3, blk3, 0)
        pltpu.make_async_copy(bufs[(nblk - 1) % 3],
                              acc.at[idxr2.at[nblk - 1]],
                              ssem[(nblk - 1) % 3]).wait()
        plsc.subcore_barrier()
        if jl + 1 < CPC:
            xcn = chunk_view(jl + 1)
            pltpu.async_copy(xcn.at[idxc2.at[0]], bufs[0], gsem[0])
            pltpu.async_copy(xcn.at[idxc2.at[1]], bufs[1], gsem[1])
        pltpu.sync_copy(acc.at[pl.ds(s * ROWS_PER_TILE, ROWS_PER_TILE)],
                        out.at[j, pl.ds(s * ROWS_PER_TILE, ROWS_PER_TILE)])
        if jl + 1 < CPC:
            for z in range(ROWS_PER_TILE // K):
                pltpu.sync_copy(zbuf,
                                acc.at[pl.ds(s * ROWS_PER_TILE + z * K, K)])
        plsc.subcore_barrier()


def kernel(X, S_rows, S_cols, S_vals):
    nnz = S_rows.shape[0]
    per_tile = -(-nnz // NSUB)
    per_tile = -(-per_tile // (3 * K)) * (3 * K)
    nblk = per_tile // K
    pad = per_tile * NSUB - nnz
    rows_p = jnp.pad(S_rows, (0, pad)).reshape(NSUB, nblk, K)
    cols_p = jnp.pad(S_cols, (0, pad)).reshape(NSUB, nblk, K)
    vals_p = jnp.pad(S_vals, (0, pad)).reshape(NSUB, nblk, K)
    xc = X.reshape(NCHUNK, CW, N).transpose(0, 2, 1).reshape(NCHUNK * N, CW)

    mesh = plsc.VectorSubcoreMesh(core_axis_name="c", subcore_axis_name="s",
                                  num_cores=NCORES, num_subcores=NSUB)
    f = pl.kernel(
        functools.partial(_spmm_body, nblk),
        out_type=jax.ShapeDtypeStruct((NCHUNK, N, CW), jnp.float32),
        mesh=mesh,
        scratch_types=[
            pltpu.VMEM((nblk, K), jnp.int32),
            pltpu.VMEM((nblk, K), jnp.int32),
            pltpu.VMEM((nblk, K), jnp.float32),
            pltpu.VMEM((K, CW), jnp.float32),
            pltpu.VMEM((K, CW), jnp.float32),
            pltpu.VMEM((K, CW), jnp.float32),
            pltpu.VMEM((K, CW), jnp.float32),
            pltpu.VMEM_SHARED((N, CW), jnp.float32),
            pltpu.SemaphoreType.DMA,
            pltpu.SemaphoreType.DMA,
            pltpu.SemaphoreType.DMA,
            pltpu.SemaphoreType.DMA,
            pltpu.SemaphoreType.DMA,
            pltpu.SemaphoreType.DMA,
        ],
        compiler_params=pltpu.CompilerParams(use_tc_tiling_on_sc=False),
    )
    out_c = f(xc, rows_p, cols_p, vals_p)
    return out_c.transpose(0, 2, 1).reshape(BATCH, N)
